# baseline (speedup 1.0000x reference)
import jax
import jax.numpy as jnp
from jax.experimental import pallas as pl
from jax.experimental.pallas import tpu as pltpu
from jax.experimental.pallas import tpu_sc as plsc

B = 4096
L = 200
D = 64
DP = 128
N = B * L
V = 1000000

ROWS_PER_STEP = 4
W = ROWS_PER_STEP * L
GATHER_CHUNK = 80
NUM_CHUNKS = W // GATHER_CHUNK


def kernel(inputs, masks, table):
    mesh = plsc.VectorSubcoreMesh(core_axis_name="core", subcore_axis_name="subcore")

    @pl.kernel(
        out_type=jax.ShapeDtypeStruct((N, DP), table.dtype),
        mesh=mesh,
        compiler_params=pltpu.CompilerParams(
            use_tc_tiling_on_sc=False, needs_layout_passes=False
        ),
        scratch_types=[pltpu.SemaphoreType.DMA],
    )
    def gather_kernel(table_hbm, idx_hbm, out_hbm, sem):
        def body(i_vmem, o_vmem):
            handles = [
                pltpu.async_copy(
                    table_hbm.at[i_vmem.at[0, pl.ds(k * GATHER_CHUNK, GATHER_CHUNK)]],
                    o_vmem.at[pl.ds(k * GATHER_CHUNK, GATHER_CHUNK)],
                    sem,
                )
                for k in range(NUM_CHUNKS)
            ]
            for h in handles:
                h.wait()

        pltpu.emit_pipeline(
            body,
            grid=(B // ROWS_PER_STEP,),
            in_specs=[pl.BlockSpec((1, W), index_map=lambda i: (i, 0))],
            out_specs=[pl.BlockSpec((W, D), index_map=lambda i: (i, 0))],
            core_axis_name=("core", "subcore"),
            dimension_semantics=(pltpu.PARALLEL,),
        )(idx_hbm, out_hbm)

    table_wide = jnp.pad(table, ((0, 0), (0, DP - D))).reshape(2 * V, D)
    out_wide = gather_kernel(table_wide, inputs.reshape(B // ROWS_PER_STEP, W) * 2)
    return out_wide[:, :D].reshape(B, L, D)
